# baseline (speedup 1.0000x reference)
import functools

import jax
import jax.numpy as jnp
from jax.experimental import pallas as pl
from jax.experimental.pallas import tpu as pltpu

B = 4
N = 16384
S = 1024
K = 8
R = 128
C = 128


def _argmax5(d, i, x, y, z):
    def comb(a, b):
        da, ia, xa, ya, za = a
        db, ib, xb, yb, zb = b
        take = (db > da) | ((db == da) & (ib < ia))
        return (jnp.where(take, db, da), jnp.where(take, ib, ia),
                jnp.where(take, xb, xa), jnp.where(take, yb, ya),
                jnp.where(take, zb, za))

    v = (d, i, x, y, z)
    rows = v[0].shape[0]
    while rows > 8:
        rows //= 2
        v = comb(tuple(a[:rows] for a in v), tuple(a[rows:] for a in v))
    for s in (4, 2, 1):
        v = comb(v, tuple(jnp.roll(a, s, axis=0) for a in v))
    for s in (64, 32, 16, 8, 4, 2, 1):
        v = comb(v, tuple(jnp.roll(a, s, axis=1) for a in v))
    return v


def _fps_body(xs_ref, ys_ref, zs_ref, idx_ref, qx_ref, qy_ref, qz_ref):
    X = [xs_ref[b] for b in range(B)]
    Y = [ys_ref[b] for b in range(B)]
    Z = [zs_ref[b] for b in range(B)]
    flat = (jax.lax.broadcasted_iota(jnp.int32, (R, C), 0) * C
            + jax.lax.broadcasted_iota(jnp.int32, (R, C), 1))
    step_flat = (jax.lax.broadcasted_iota(jnp.int32, (8, 128), 0) * 128
                 + jax.lax.broadcasted_iota(jnp.int32, (8, 128), 1))

    def body(t, carry):
        out = []
        for b in range(B):
            dist, curv, cxv, cyv, czv, acc_i, aqx, aqy, aqz = carry[b]
            emit = step_flat == t
            acc_i = jnp.where(emit, curv, acc_i)
            aqx = jnp.where(emit, cxv, aqx)
            aqy = jnp.where(emit, cyv, aqy)
            aqz = jnp.where(emit, czv, aqz)
            cx = jnp.broadcast_to(cxv[0:1, :], (R, C))
            cy = jnp.broadcast_to(cyv[0:1, :], (R, C))
            cz = jnp.broadcast_to(czv[0:1, :], (R, C))
            dx = X[b] - cx
            dy = Y[b] - cy
            dz = Z[b] - cz
            d = dx * dx + dy * dy + dz * dz
            dist = jnp.minimum(dist, d)
            mv, iv, nx, ny, nz = _argmax5(dist, flat, X[b], Y[b], Z[b])
            out.append((dist, iv, nx, ny, nz, acc_i, aqx, aqy, aqz))
        return tuple(out)

    zero8 = jnp.zeros((8, 128), jnp.float32)
    inits = []
    for b in range(B):
        hit = (step_flat == 0)
        c0x = jnp.sum(jnp.where(hit, X[b][0:8], 0.0))
        c0y = jnp.sum(jnp.where(hit, Y[b][0:8], 0.0))
        c0z = jnp.sum(jnp.where(hit, Z[b][0:8], 0.0))
        inits.append((jnp.full((R, C), 1e10, jnp.float32),
                      jnp.zeros((8, 128), jnp.int32),
                      jnp.full((8, 128), 1.0) * c0x,
                      jnp.full((8, 128), 1.0) * c0y,
                      jnp.full((8, 128), 1.0) * c0z,
                      jnp.zeros((8, 128), jnp.int32), zero8, zero8, zero8))
    fin = jax.lax.fori_loop(0, S, body, tuple(inits))
    for b in range(B):
        _, _, _, _, _, acc_i, aqx, aqy, aqz = fin[b]
        idx_ref[b] = acc_i
        qx_ref[b] = aqx
        qy_ref[b] = aqy
        qz_ref[b] = aqz


def _bf16_rtne(x):
    u = jax.lax.bitcast_convert_type(x, jnp.uint32)
    r = (u + 0x7FFF + ((u >> 16) & 1)) & jnp.uint32(0xFFFF0000)
    return jax.lax.bitcast_convert_type(r, jnp.float32)


def _knn_body(qcx_ref, qcy_ref, qcz_ref, px_ref, py_ref, pz_ref, out_ref,
              pxb_ref, pyb_ref, pzb_ref, pp_ref):
    qg = pl.program_id(1)

    @pl.when(qg == 0)
    def _():
        PXf = jnp.broadcast_to(px_ref[0], (8, N))
        PYf = jnp.broadcast_to(py_ref[0], (8, N))
        PZf = jnp.broadcast_to(pz_ref[0], (8, N))
        pxb_ref[...] = _bf16_rtne(PXf)
        pyb_ref[...] = _bf16_rtne(PYf)
        pzb_ref[...] = _bf16_rtne(PZf)
        pp_ref[...] = PXf * PXf + PYf * PYf + PZf * PZf
    lane = jax.lax.broadcasted_iota(jnp.int32, (8, N), 1)
    lane8 = jax.lax.broadcasted_iota(jnp.int32, (8, 8), 1)

    def group_dist(g):
        qxc = qcx_ref[0, g]
        qyc = qcy_ref[0, g]
        qzc = qcz_ref[0, g]
        qq = qxc * qxc + qyc * qyc + qzc * qzc
        qp = (_bf16_rtne(qxc) * pxb_ref[...]
              + _bf16_rtne(qyc) * pyb_ref[...]
              + _bf16_rtne(qzc) * pzb_ref[...])
        return qq + pp_ref[...] - 2.0 * qp

    NG = 8
    dists = [group_dist(g) for g in range(NG)]
    accs = [jnp.zeros((8, 8), jnp.int32) for _ in range(NG)]
    for k in range(K):
        for g in range(NG):
            dist = dists[g]
            m = jnp.min(dist, axis=1, keepdims=True)
            cand = jnp.where(dist == m, lane, jnp.int32(0x3FFFFFFF))
            idxk = jnp.min(cand, axis=1, keepdims=True)
            accs[g] = jnp.where(lane8 == k, idxk, accs[g])
            dists[g] = jnp.where(lane == idxk, jnp.float32(jnp.inf), dist)
    for g in range(NG):
        out_ref[0, 8 * g:8 * (g + 1)] = accs[g]


def _mlp_body(x_ref, y_ref, z_ref, w1_ref, b1_ref, w2_ref, b2_ref, o_ref):
    x = x_ref[...]
    y = y_ref[...]
    z = z_ref[...]
    w1x = w1_ref[0:1, :]
    w1y = w1_ref[1:2, :]
    w1z = w1_ref[2:3, :]
    h = x * w1x + y * w1y + z * w1z + b1_ref[0:1, :]
    h = jnp.maximum(h, 0.0)
    o = jnp.dot(h, w2_ref[...], preferred_element_type=jnp.float32)
    o_ref[...] = o + b2_ref[0:1, :]


def kernel(point_cloud, W1, b1, W2, b2):
    xs = point_cloud[:, :, 0]
    ys = point_cloud[:, :, 1]
    zs = point_cloud[:, :, 2]
    xsq = xs.reshape(B, R, C)
    ysq = ys.reshape(B, R, C)
    zsq = zs.reshape(B, R, C)

    grid_fps = pl.GridSpec(
        grid=(1,),
        in_specs=[pl.BlockSpec((B, R, C), lambda i: (0, 0, 0))] * 3,
        out_specs=[pl.BlockSpec((B, 8, 128), lambda i: (0, 0, 0))] * 4,
    )
    fps_i, qx, qy, qz = pl.pallas_call(
        _fps_body,
        grid_spec=grid_fps,
        out_shape=[
            jax.ShapeDtypeStruct((B, 8, 128), jnp.int32),
            jax.ShapeDtypeStruct((B, 8, 128), jnp.float32),
            jax.ShapeDtypeStruct((B, 8, 128), jnp.float32),
            jax.ShapeDtypeStruct((B, 8, 128), jnp.float32),
        ],
    )(xsq, ysq, zsq)

    xr = xs.reshape(B, 1, N)
    yr = ys.reshape(B, 1, N)
    zr = zs.reshape(B, 1, N)
    qcx = qx.reshape(B, S // 8, 8, 1)
    qcy = qy.reshape(B, S // 8, 8, 1)
    qcz = qz.reshape(B, S // 8, 8, 1)
    knn_idx = pl.pallas_call(
        _knn_body,
        grid=(B, S // 64),
        in_specs=[pl.BlockSpec((1, 8, 8, 1), lambda b, q: (b, q, 0, 0))] * 3
        + [pl.BlockSpec((1, 1, N), lambda b, q: (b, 0, 0))] * 3,
        out_specs=pl.BlockSpec((1, 64, 8), lambda b, q: (b, q, 0)),
        out_shape=jax.ShapeDtypeStruct((B, S, K), jnp.int32),
        scratch_shapes=[pltpu.VMEM((8, N), jnp.float32)] * 4,
    )(qcx, qcy, qcz, xr, yr, zr)

    BLK = 4096
    xcol = xs.reshape(B * N, 1)
    ycol = ys.reshape(B * N, 1)
    zcol = zs.reshape(B * N, 1)
    grid_mlp = pl.GridSpec(
        grid=(B * N // BLK,),
        in_specs=[pl.BlockSpec((BLK, 1), lambda i: (i, 0))] * 3
        + [pl.BlockSpec((3, 64), lambda i: (0, 0)),
           pl.BlockSpec((1, 64), lambda i: (0, 0)),
           pl.BlockSpec((64, 3), lambda i: (0, 0)),
           pl.BlockSpec((1, 3), lambda i: (0, 0))],
        out_specs=pl.BlockSpec((BLK, 3), lambda i: (i, 0)),
    )
    out = pl.pallas_call(
        _mlp_body,
        grid_spec=grid_mlp,
        out_shape=jax.ShapeDtypeStruct((B * N, 3), jnp.float32),
    )(xcol, ycol, zcol, W1, b1.reshape(1, 64), W2, b2.reshape(1, 3))

    fps_idx = fps_i.reshape(B, S)
    return out.reshape(B, N, 3), knn_idx, fps_idx


# speedup vs baseline: 1.0245x; 1.0245x over previous
import functools

import jax
import jax.numpy as jnp
from jax.experimental import pallas as pl
from jax.experimental.pallas import tpu as pltpu

B = 4
N = 16384
S = 1024
K = 8
R = 128
C = 128


def _argmax5(d, i, x, y, z):
    def comb(a, b):
        da, ia, xa, ya, za = a
        db, ib, xb, yb, zb = b
        take = (db > da) | ((db == da) & (ib < ia))
        return (jnp.where(take, db, da), jnp.where(take, ib, ia),
                jnp.where(take, xb, xa), jnp.where(take, yb, ya),
                jnp.where(take, zb, za))

    v = (d, i, x, y, z)
    rows = v[0].shape[0]
    while rows > 8:
        rows //= 2
        v = comb(tuple(a[:rows] for a in v), tuple(a[rows:] for a in v))
    for s in (4, 2, 1):
        v = comb(v, tuple(jnp.roll(a, s, axis=0) for a in v))
    for s in (64, 32, 16, 8, 4, 2, 1):
        v = comb(v, tuple(jnp.roll(a, s, axis=1) for a in v))
    return v


def _fps_body(xs_ref, ys_ref, zs_ref, idx_ref, qx_ref, qy_ref, qz_ref):
    X = [xs_ref[b] for b in range(B)]
    Y = [ys_ref[b] for b in range(B)]
    Z = [zs_ref[b] for b in range(B)]
    flat = (jax.lax.broadcasted_iota(jnp.int32, (R, C), 0) * C
            + jax.lax.broadcasted_iota(jnp.int32, (R, C), 1))
    step_flat = (jax.lax.broadcasted_iota(jnp.int32, (8, 128), 0) * 128
                 + jax.lax.broadcasted_iota(jnp.int32, (8, 128), 1))

    def body(t, carry):
        out = []
        for b in range(B):
            dist, curv, cxv, cyv, czv, acc_i, aqx, aqy, aqz = carry[b]
            emit = step_flat == t
            acc_i = jnp.where(emit, curv, acc_i)
            aqx = jnp.where(emit, cxv, aqx)
            aqy = jnp.where(emit, cyv, aqy)
            aqz = jnp.where(emit, czv, aqz)
            cx = jnp.broadcast_to(cxv[0:1, :], (R, C))
            cy = jnp.broadcast_to(cyv[0:1, :], (R, C))
            cz = jnp.broadcast_to(czv[0:1, :], (R, C))
            dx = X[b] - cx
            dy = Y[b] - cy
            dz = Z[b] - cz
            d = dx * dx + dy * dy + dz * dz
            dist = jnp.minimum(dist, d)
            mv, iv, nx, ny, nz = _argmax5(dist, flat, X[b], Y[b], Z[b])
            out.append((dist, iv, nx, ny, nz, acc_i, aqx, aqy, aqz))
        return tuple(out)

    zero8 = jnp.zeros((8, 128), jnp.float32)
    inits = []
    for b in range(B):
        hit = (step_flat == 0)
        c0x = jnp.sum(jnp.where(hit, X[b][0:8], 0.0))
        c0y = jnp.sum(jnp.where(hit, Y[b][0:8], 0.0))
        c0z = jnp.sum(jnp.where(hit, Z[b][0:8], 0.0))
        inits.append((jnp.full((R, C), 1e10, jnp.float32),
                      jnp.zeros((8, 128), jnp.int32),
                      jnp.full((8, 128), 1.0) * c0x,
                      jnp.full((8, 128), 1.0) * c0y,
                      jnp.full((8, 128), 1.0) * c0z,
                      jnp.zeros((8, 128), jnp.int32), zero8, zero8, zero8))
    fin = jax.lax.fori_loop(0, S, body, tuple(inits))
    for b in range(B):
        _, _, _, _, _, acc_i, aqx, aqy, aqz = fin[b]
        idx_ref[b] = acc_i
        qx_ref[b] = aqx
        qy_ref[b] = aqy
        qz_ref[b] = aqz


def _bf16_rtne(x):
    u = jax.lax.bitcast_convert_type(x, jnp.uint32)
    r = (u + 0x7FFF + ((u >> 16) & 1)) & jnp.uint32(0xFFFF0000)
    return jax.lax.bitcast_convert_type(r, jnp.float32)


def _knn_body(qcx_ref, qcy_ref, qcz_ref, px_ref, py_ref, pz_ref, out_ref,
              pxb_ref, pyb_ref, pzb_ref, pp_ref):
    qg = pl.program_id(1)

    @pl.when(qg == 0)
    def _():
        PXf = jnp.broadcast_to(px_ref[0], (8, N))
        PYf = jnp.broadcast_to(py_ref[0], (8, N))
        PZf = jnp.broadcast_to(pz_ref[0], (8, N))
        pxb_ref[...] = _bf16_rtne(PXf)
        pyb_ref[...] = _bf16_rtne(PYf)
        pzb_ref[...] = _bf16_rtne(PZf)
        pp_ref[...] = PXf * PXf + PYf * PYf + PZf * PZf
    lane = jax.lax.broadcasted_iota(jnp.int32, (8, N), 1)
    lane8 = jax.lax.broadcasted_iota(jnp.int32, (8, 8), 1)

    def group_dist(g):
        qxc = qcx_ref[0, g]
        qyc = qcy_ref[0, g]
        qzc = qcz_ref[0, g]
        qq = qxc * qxc + qyc * qyc + qzc * qzc
        qp = (_bf16_rtne(qxc) * pxb_ref[...]
              + _bf16_rtne(qyc) * pyb_ref[...]
              + _bf16_rtne(qzc) * pzb_ref[...])
        return qq + pp_ref[...] - 2.0 * qp

    NG = 16
    dists = [group_dist(g) for g in range(NG)]
    accs = [jnp.zeros((8, 8), jnp.int32) for _ in range(NG)]
    for k in range(K):
        for g in range(NG):
            dist = dists[g]
            m = jnp.min(dist, axis=1, keepdims=True)
            cand = jnp.where(dist == m, lane, jnp.int32(0x3FFFFFFF))
            idxk = jnp.min(cand, axis=1, keepdims=True)
            accs[g] = jnp.where(lane8 == k, idxk, accs[g])
            dists[g] = jnp.where(lane == idxk, jnp.float32(jnp.inf), dist)
    for g in range(NG):
        out_ref[0, 8 * g:8 * (g + 1)] = accs[g]


def _mlp_body(x_ref, y_ref, z_ref, w1_ref, b1_ref, w2_ref, b2_ref, o_ref):
    x = x_ref[...]
    y = y_ref[...]
    z = z_ref[...]
    w1x = w1_ref[0:1, :]
    w1y = w1_ref[1:2, :]
    w1z = w1_ref[2:3, :]
    h = x * w1x + y * w1y + z * w1z + b1_ref[0:1, :]
    h = jnp.maximum(h, 0.0)
    o = jnp.dot(h, w2_ref[...], preferred_element_type=jnp.float32)
    o_ref[...] = o + b2_ref[0:1, :]


def kernel(point_cloud, W1, b1, W2, b2):
    xs = point_cloud[:, :, 0]
    ys = point_cloud[:, :, 1]
    zs = point_cloud[:, :, 2]
    xsq = xs.reshape(B, R, C)
    ysq = ys.reshape(B, R, C)
    zsq = zs.reshape(B, R, C)

    grid_fps = pl.GridSpec(
        grid=(1,),
        in_specs=[pl.BlockSpec((B, R, C), lambda i: (0, 0, 0))] * 3,
        out_specs=[pl.BlockSpec((B, 8, 128), lambda i: (0, 0, 0))] * 4,
    )
    fps_i, qx, qy, qz = pl.pallas_call(
        _fps_body,
        grid_spec=grid_fps,
        out_shape=[
            jax.ShapeDtypeStruct((B, 8, 128), jnp.int32),
            jax.ShapeDtypeStruct((B, 8, 128), jnp.float32),
            jax.ShapeDtypeStruct((B, 8, 128), jnp.float32),
            jax.ShapeDtypeStruct((B, 8, 128), jnp.float32),
        ],
    )(xsq, ysq, zsq)

    xr = xs.reshape(B, 1, N)
    yr = ys.reshape(B, 1, N)
    zr = zs.reshape(B, 1, N)
    qcx = qx.reshape(B, S // 8, 8, 1)
    qcy = qy.reshape(B, S // 8, 8, 1)
    qcz = qz.reshape(B, S // 8, 8, 1)
    knn_idx = pl.pallas_call(
        _knn_body,
        grid=(B, S // 128),
        in_specs=[pl.BlockSpec((1, 16, 8, 1), lambda b, q: (b, q, 0, 0))] * 3
        + [pl.BlockSpec((1, 1, N), lambda b, q: (b, 0, 0))] * 3,
        out_specs=pl.BlockSpec((1, 128, 8), lambda b, q: (b, q, 0)),
        out_shape=jax.ShapeDtypeStruct((B, S, K), jnp.int32),
        scratch_shapes=[pltpu.VMEM((8, N), jnp.float32)] * 4,
    )(qcx, qcy, qcz, xr, yr, zr)

    BLK = 4096
    xcol = xs.reshape(B * N, 1)
    ycol = ys.reshape(B * N, 1)
    zcol = zs.reshape(B * N, 1)
    grid_mlp = pl.GridSpec(
        grid=(B * N // BLK,),
        in_specs=[pl.BlockSpec((BLK, 1), lambda i: (i, 0))] * 3
        + [pl.BlockSpec((3, 64), lambda i: (0, 0)),
           pl.BlockSpec((1, 64), lambda i: (0, 0)),
           pl.BlockSpec((64, 3), lambda i: (0, 0)),
           pl.BlockSpec((1, 3), lambda i: (0, 0))],
        out_specs=pl.BlockSpec((BLK, 3), lambda i: (i, 0)),
    )
    out = pl.pallas_call(
        _mlp_body,
        grid_spec=grid_mlp,
        out_shape=jax.ShapeDtypeStruct((B * N, 3), jnp.float32),
    )(xcol, ycol, zcol, W1, b1.reshape(1, 64), W2, b2.reshape(1, 3))

    fps_idx = fps_i.reshape(B, S)
    return out.reshape(B, N, 3), knn_idx, fps_idx
